# baseline (speedup 1.0000x reference)
import functools

import jax
import jax.numpy as jnp
from jax import lax
from jax.experimental import pallas as pl
from jax.experimental.pallas import tpu as pltpu
from jax.experimental.pallas import tpu_sc as plsc

B = 1024
L = 50
D = 300
V = 100000
LANES = 16
NCHUNK = D // LANES
TAIL = D - NCHUNK * LANES
NC = 2
NS = 16
BPW = B // (NC * NS)
WPAD = 72

_SC_PARAMS = pltpu.CompilerParams(needs_layout_passes=False,
                                  use_tc_tiling_on_sc=False)


def _lane_sum(v):
    lane = lax.iota(jnp.int32, 16)
    dn = lax.GatherDimensionNumbers(offset_dims=(), collapsed_slice_dims=(0,),
                                    start_index_map=(0,))
    for k in (1, 2, 4, 8):
        idx = lane ^ k
        v = v + lax.gather(v, idx[:, None], dn, (1,),
                           mode=lax.GatherScatterMode.PROMISE_IN_BOUNDS)
    return v


def _rsqrt_nr(x):
    i = plsc.bitcast(x, jnp.int32)
    y = plsc.bitcast(jnp.int32(0x5F3759DF) - lax.shift_right_logical(i, 1),
                     jnp.float32)
    for _ in range(4):
        y = y * (1.5 - 0.5 * x * y * y)
    return y


def _pool_body(g_hbm, w_hbm, table_hbm, out_hbm, g_v, w_v, g_item, pair_v, y_v,
               sem):
    wid = lax.axis_index("s") * NC + lax.axis_index("c")
    base = wid * BPW

    pltpu.sync_copy(g_hbm.at[pl.ds(base, BPW)], g_v)
    pltpu.sync_copy(w_hbm.at[pl.ds(base, BPW)], w_v)

    lane = lax.iota(jnp.int32, 16)
    tail_mask = lane >= (LANES - TAIL)

    def per_item(j, _):
        g_item[pl.ds(0, 16)] = g_v[j, pl.ds(0, 16)]
        g_item[pl.ds(16, 16)] = g_v[j, pl.ds(16, 16)]
        g_item[pl.ds(32, 16)] = g_v[j, pl.ds(32, 16)]
        g_item[pl.ds(L - 16, 16)] = g_v[j, pl.ds(L - 16, 16)]
        pltpu.async_copy(table_hbm.at[g_item], pair_v, sem).wait()

        def per_row(r, accs):
            off = w_v[j, pl.ds(r, 16)][0]
            chunks = [pair_v[r, pl.ds(off + 16 * c, 16)] for c in range(NCHUNK)]
            vt = pair_v[r, pl.ds(off + D - 16, 16)]
            vt = jnp.where(tail_mask, vt, 0.0)
            s = vt * vt
            for v in chunks:
                s = s + v * v
            nsq = jnp.maximum(_lane_sum(s), 1e-14)
            scale = jnp.minimum(_rsqrt_nr(nsq), 1.0)
            scale = scale * (1.0 / L)
            new = [a + scale * v for a, v in zip(accs[:-1], chunks)]
            new.append(accs[-1] + scale * vt)
            return tuple(new)

        zero = jnp.zeros((16,), jnp.float32)
        accs = lax.fori_loop(0, L, per_row, tuple(zero for _ in range(NCHUNK + 1)))
        y_v[j, pl.ds(D - 16, 16)] = accs[-1]
        for c in range(NCHUNK):
            y_v[j, pl.ds(16 * c, 16)] = accs[c]
        return _

    lax.fori_loop(0, BPW, per_item, 0)
    pltpu.sync_copy(y_v, out_hbm.at[pl.ds(base, BPW)])


def _pool(gidx, woff, table2):
    mesh = plsc.VectorSubcoreMesh(core_axis_name="c", subcore_axis_name="s")
    kern = functools.partial(
        pl.kernel,
        out_type=jax.ShapeDtypeStruct((B, D), jnp.float32),
        mesh=mesh,
        compiler_params=_SC_PARAMS,
        scratch_types=[
            pltpu.VMEM((BPW, L), jnp.int32),
            pltpu.VMEM((BPW, WPAD), jnp.int32),
            pltpu.VMEM((L,), jnp.int32),
            pltpu.VMEM((L, 2 * D), jnp.float32),
            pltpu.VMEM((BPW, D), jnp.float32),
            pltpu.SemaphoreType.DMA,
        ],
    )(_pool_body)
    return kern(gidx, woff, table2)


def _project_body(x_ref, w_ref, b_ref, o_ref):
    xb = x_ref[...].astype(jnp.bfloat16)
    wb = w_ref[...].astype(jnp.bfloat16)
    acc = lax.dot_general(xb, wb, (((1,), (1,)), ((), ())),
                          preferred_element_type=jnp.float32)
    o_ref[...] = acc + b_ref[...]


def _project(x, w, b2):
    tv = 2048
    grid = pl.cdiv(V, tv)
    return pl.pallas_call(
        _project_body,
        grid=(grid,),
        in_specs=[
            pl.BlockSpec((B, D), lambda i: (0, 0)),
            pl.BlockSpec((tv, D), lambda i: (i, 0)),
            pl.BlockSpec((1, tv), lambda i: (0, i)),
        ],
        out_specs=pl.BlockSpec((B, tv), lambda i: (0, i)),
        out_shape=jax.ShapeDtypeStruct((B, V), jnp.float32),
    )(x, w, b2)


def kernel(input_word_ids, emb_table, W, b):
    idx = input_word_ids.astype(jnp.int32)
    gidx = lax.shift_right_logical(idx, 1)
    woff = (idx & 1) * D
    woff = jnp.pad(woff, ((0, 0), (0, WPAD - L)))
    table2 = emb_table.reshape(V // 2, 2 * D)
    x = _pool(gidx, woff, table2)
    return _project(x, W, b.reshape(1, V))
